# baseline (speedup 1.0000x reference)
import functools

import jax
import jax.numpy as jnp
from jax import lax
from jax.experimental import pallas as pl
from jax.experimental.pallas import tpu as pltpu
from jax.experimental.pallas import tpu_sc as plsc

_NC = 2
_NS = 16
_NW = _NC * _NS


def _router_body(cap, nslot, x_ref, rw_ref, logits_ref, scaled_ref,
                 sidx_ref, gidx_ref, fei_ref, carry_ref):
    i = pl.program_id(0)

    @pl.when(i == 0)
    def _():
        carry_ref[...] = jnp.zeros_like(carry_ref)

    x = x_ref[0]
    c_rows, e = logits_ref.shape
    logits = lax.dot_general(x, rw_ref[...], (((1,), (1,)), ((), ())),
                             preferred_element_type=jnp.float32)
    logits_ref[...] = logits
    m = jnp.max(logits, axis=1, keepdims=True)
    lane = lax.broadcasted_iota(jnp.int32, (c_rows, e), 1)
    eidx = jnp.min(jnp.where(logits == m, lane, e), axis=1, keepdims=True)
    p = 1.0 / jnp.sum(jnp.exp(logits - m), axis=1, keepdims=True)
    onehot = (lane == eidx).astype(jnp.float32)
    r_i = lax.broadcasted_iota(jnp.int32, (c_rows, c_rows), 0)
    c_i = lax.broadcasted_iota(jnp.int32, (c_rows, c_rows), 1)
    tril = (r_i >= c_i).astype(jnp.float32)
    prio = jnp.dot(tril, onehot, preferred_element_type=jnp.float32)
    prio = prio + carry_ref[0:1, :]
    carry_ref[0:1, :] += jnp.sum(onehot, axis=0, keepdims=True)
    prio_t = jnp.sum(onehot * prio, axis=1, keepdims=True)
    kept = prio_t <= float(cap)
    tok = i * c_rows + lax.broadcasted_iota(jnp.int32, (c_rows, 1), 0)
    slot = eidx * cap + prio_t.astype(jnp.int32) - 1
    sidx_c = jnp.where(kept, slot, 2 * nslot + tok).astype(jnp.float32)
    gidx_c = jnp.where(kept, nslot + slot, 2 * nslot + tok).astype(jnp.float32)
    fei_c = jnp.where(kept, eidx, 0).astype(jnp.float32)
    cols = jnp.concatenate([sidx_c, gidx_c, fei_c], axis=1)
    eye = (r_i == c_i).astype(jnp.float32)
    colsT = lax.dot_general(cols, eye, (((0,), (0,)), ((), ())),
                            precision=lax.Precision.HIGHEST,
                            preferred_element_type=jnp.float32)
    sidx_ref[pl.ds(i * c_rows, c_rows)] = colsT[0].astype(jnp.int32)
    gidx_ref[pl.ds(i * c_rows, c_rows)] = colsT[1].astype(jnp.int32)
    fei_ref[pl.ds(i * c_rows, c_rows)] = colsT[2].astype(jnp.int32)
    scaled_ref[...] = x * p


def _ffn_body(nblk, x_ref, wi_ref, wo_ref, y_ref):
    h = jnp.maximum(
        jnp.dot(x_ref[...], wi_ref[0], preferred_element_type=jnp.float32),
        0.0)
    y_ref[...] = jnp.dot(h, wo_ref[0], preferred_element_type=jnp.float32)


def kernel(hidden_states, router_w, wi, wo):
    b, s, d = hidden_states.shape
    e = router_w.shape[1]
    d_ff = wi.shape[2]
    cap = 40
    nslot = e * cap
    n_tok = b * s
    nrow = 2 * nslot + n_tok
    chunk = 512
    n_chunks = n_tok // chunk
    tpw = n_tok // _NW

    logits, scaled, sidx, gidx, fei = pl.pallas_call(
        functools.partial(_router_body, cap, nslot),
        grid=(n_chunks,),
        in_specs=[
            pl.BlockSpec((1, chunk, d), lambda i: (0, i, 0)),
            pl.BlockSpec((e, d), lambda i: (0, 0)),
        ],
        out_specs=[
            pl.BlockSpec((chunk, e), lambda i: (i, 0)),
            pl.BlockSpec((chunk, d), lambda i: (i, 0)),
            pl.BlockSpec((n_tok,), lambda i: (0,)),
            pl.BlockSpec((n_tok,), lambda i: (0,)),
            pl.BlockSpec((n_tok,), lambda i: (0,)),
        ],
        out_shape=[
            jax.ShapeDtypeStruct((n_tok, e), jnp.float32),
            jax.ShapeDtypeStruct((n_tok, d), jnp.float32),
            jax.ShapeDtypeStruct((n_tok,), jnp.int32),
            jax.ShapeDtypeStruct((n_tok,), jnp.int32),
            jax.ShapeDtypeStruct((n_tok,), jnp.int32),
        ],
        scratch_shapes=[pltpu.VMEM((8, e), jnp.float32)],
    )(hidden_states, router_w.T)

    sidx_flat = sidx
    gidx_flat = gidx

    mesh = plsc.VectorSubcoreMesh(core_axis_name="c", subcore_axis_name="s")

    nch = 4
    hh = tpw // nch

    @functools.partial(
        pl.kernel,
        out_type=jax.ShapeDtypeStruct((nrow, d), jnp.float32),
        mesh=mesh,
        scratch_types=[
            pltpu.VMEM((tpw, d), jnp.float32),
            [pltpu.VMEM((hh,), jnp.int32) for _ in range(nch)],
            [pltpu.SemaphoreType.DMA for _ in range(nch)],
            pltpu.SemaphoreType.DMA,
        ],
    )
    def sc_scatter(scaled_hbm, sidx_hbm, buf_hbm, rows_v, idxs_v, sems_i, sem_o):
        wid = lax.axis_index("s") * _NC + lax.axis_index("c")
        base = wid * tpw
        for k in range(nch):
            pltpu.sync_copy(sidx_hbm.at[pl.ds(base + k * hh, hh)], idxs_v[k])
        ins = [pltpu.async_copy(scaled_hbm.at[pl.ds(base + k * hh, hh)],
                                rows_v.at[pl.ds(k * hh, hh)], sems_i[k])
               for k in range(nch)]
        outs = []
        for k in range(nch):
            ins[k].wait()
            outs.append(pltpu.async_copy(rows_v.at[pl.ds(k * hh, hh)],
                                         buf_hbm.at[idxs_v[k]], sem_o))
        for o in outs:
            o.wait()

    buf = sc_scatter(scaled, sidx_flat)

    nblk = nrow // cap
    buf = pl.pallas_call(
        functools.partial(_ffn_body, nblk),
        grid=(e,),
        in_specs=[
            pl.BlockSpec((cap, d), lambda i: (i, 0)),
            pl.BlockSpec((1, d, d_ff), lambda i: (i, 0, 0)),
            pl.BlockSpec((1, d_ff, d), lambda i: (i, 0, 0)),
        ],
        out_specs=pl.BlockSpec((cap, d), lambda i: (e + i, 0)),
        out_shape=jax.ShapeDtypeStruct((nrow, d), jnp.float32),
        input_output_aliases={0: 0},
    )(buf, wi, wo)

    @functools.partial(
        pl.kernel,
        out_type=jax.ShapeDtypeStruct((n_tok, d), jnp.float32),
        mesh=mesh,
        scratch_types=[
            pltpu.VMEM((tpw, d), jnp.float32),
            [pltpu.VMEM((hh,), jnp.int32) for _ in range(nch)],
            [pltpu.SemaphoreType.DMA for _ in range(nch)],
            pltpu.SemaphoreType.DMA,
        ],
    )
    def sc_gather(buf_hbm, gidx_hbm, out_hbm, rows_v, idxs_v, sems_i, sem_o):
        wid = lax.axis_index("s") * _NC + lax.axis_index("c")
        base = wid * tpw
        for k in range(nch):
            pltpu.sync_copy(gidx_hbm.at[pl.ds(base + k * hh, hh)], idxs_v[k])
        ins = [pltpu.async_copy(buf_hbm.at[idxs_v[k]],
                                rows_v.at[pl.ds(k * hh, hh)], sems_i[k])
               for k in range(nch)]
        outs = []
        for k in range(nch):
            ins[k].wait()
            outs.append(pltpu.async_copy(rows_v.at[pl.ds(k * hh, hh)],
                                         out_hbm.at[pl.ds(base + k * hh, hh)],
                                         sem_o))
        for o in outs:
            o.wait()

    out_flat = sc_gather(buf, gidx_flat)

    out = out_flat.reshape(b, s, d)
    router_logits = logits.reshape(b, s, e)
    final_expert_index = fei.reshape(b, s)
    return out, router_logits, final_expert_index


# speedup vs baseline: 1.0118x; 1.0118x over previous
import functools

import jax
import jax.numpy as jnp
from jax import lax
from jax.experimental import pallas as pl
from jax.experimental.pallas import tpu as pltpu
from jax.experimental.pallas import tpu_sc as plsc

_NC = 2
_NS = 16
_NW = _NC * _NS


def _router_body(cap, nslot, x_ref, rw_ref, logits_ref, scaled_ref,
                 sidx_ref, gidx_ref, fei_ref, carry_ref):
    i = pl.program_id(0)

    @pl.when(i == 0)
    def _():
        carry_ref[...] = jnp.zeros_like(carry_ref)

    x = x_ref[0]
    c_rows, e = logits_ref.shape
    logits = lax.dot_general(x, rw_ref[...], (((1,), (1,)), ((), ())),
                             preferred_element_type=jnp.float32)
    logits_ref[...] = logits
    m = jnp.max(logits, axis=1, keepdims=True)
    lane = lax.broadcasted_iota(jnp.int32, (c_rows, e), 1)
    eidx = jnp.min(jnp.where(logits == m, lane, e), axis=1, keepdims=True)
    p = 1.0 / jnp.sum(jnp.exp(logits - m), axis=1, keepdims=True)
    onehot = (lane == eidx).astype(jnp.float32)
    r_i = lax.broadcasted_iota(jnp.int32, (c_rows, c_rows), 0)
    c_i = lax.broadcasted_iota(jnp.int32, (c_rows, c_rows), 1)
    tril = (r_i >= c_i).astype(jnp.float32)
    prio = jnp.dot(tril, onehot, preferred_element_type=jnp.float32)
    prio = prio + carry_ref[0:1, :]
    carry_ref[0:1, :] += jnp.sum(onehot, axis=0, keepdims=True)
    prio_t = jnp.sum(onehot * prio, axis=1, keepdims=True)
    kept = prio_t <= float(cap)
    tok = i * c_rows + lax.broadcasted_iota(jnp.int32, (c_rows, 1), 0)
    slot = eidx * cap + prio_t.astype(jnp.int32) - 1
    sidx_c = jnp.where(kept, slot, 2 * nslot + tok).astype(jnp.float32)
    gidx_c = jnp.where(kept, nslot + slot, 2 * nslot + tok).astype(jnp.float32)
    fei_c = jnp.where(kept, eidx, 0).astype(jnp.float32)
    cols = jnp.concatenate([sidx_c, gidx_c, fei_c], axis=1)
    eye = (r_i == c_i).astype(jnp.float32)
    colsT = lax.dot_general(cols, eye, (((0,), (0,)), ((), ())),
                            precision=lax.Precision.HIGHEST,
                            preferred_element_type=jnp.float32)
    sidx_ref[pl.ds(i * c_rows, c_rows)] = colsT[0].astype(jnp.int32)
    gidx_ref[pl.ds(i * c_rows, c_rows)] = colsT[1].astype(jnp.int32)
    fei_ref[pl.ds(i * c_rows, c_rows)] = colsT[2].astype(jnp.int32)
    scaled_ref[...] = x * p


def _ffn_body(nblk, x_ref, wi_ref, wo_ref, y_ref):
    h = jnp.maximum(
        jnp.dot(x_ref[...], wi_ref[0], preferred_element_type=jnp.float32),
        0.0)
    y_ref[...] = jnp.dot(h, wo_ref[0], preferred_element_type=jnp.float32)


def kernel(hidden_states, router_w, wi, wo):
    b, s, d = hidden_states.shape
    e = router_w.shape[1]
    d_ff = wi.shape[2]
    cap = 40
    nslot = e * cap
    n_tok = b * s
    nrow = 2 * nslot + n_tok
    chunk = 512
    n_chunks = n_tok // chunk
    tpw = n_tok // _NW

    logits, scaled, sidx, gidx, fei = pl.pallas_call(
        functools.partial(_router_body, cap, nslot),
        grid=(n_chunks,),
        in_specs=[
            pl.BlockSpec((1, chunk, d), lambda i: (0, i, 0)),
            pl.BlockSpec((e, d), lambda i: (0, 0)),
        ],
        out_specs=[
            pl.BlockSpec((chunk, e), lambda i: (i, 0)),
            pl.BlockSpec((chunk, d), lambda i: (i, 0)),
            pl.BlockSpec((n_tok,), lambda i: (0,)),
            pl.BlockSpec((n_tok,), lambda i: (0,)),
            pl.BlockSpec((n_tok,), lambda i: (0,)),
        ],
        out_shape=[
            jax.ShapeDtypeStruct((n_tok, e), jnp.float32),
            jax.ShapeDtypeStruct((n_tok, d), jnp.float32),
            jax.ShapeDtypeStruct((n_tok,), jnp.int32),
            jax.ShapeDtypeStruct((n_tok,), jnp.int32),
            jax.ShapeDtypeStruct((n_tok,), jnp.int32),
        ],
        scratch_shapes=[pltpu.VMEM((8, e), jnp.float32)],
    )(hidden_states, router_w.T)

    sidx_flat = sidx
    gidx_flat = gidx

    mesh = plsc.VectorSubcoreMesh(core_axis_name="c", subcore_axis_name="s")

    @functools.partial(
        pl.kernel,
        out_type=jax.ShapeDtypeStruct((nrow, d), jnp.float32),
        mesh=mesh,
        scratch_types=[
            pltpu.VMEM((tpw, d), jnp.float32),
            pltpu.VMEM((tpw,), jnp.int32),
            pltpu.SemaphoreType.DMA,
        ],
    )
    def sc_scatter(scaled_hbm, sidx_hbm, buf_hbm, rows_v, idx_v, sem):
        wid = lax.axis_index("s") * _NC + lax.axis_index("c")
        base = wid * tpw
        pltpu.sync_copy(scaled_hbm.at[pl.ds(base, tpw)], rows_v)
        pltpu.sync_copy(sidx_hbm.at[pl.ds(base, tpw)], idx_v)
        pltpu.async_copy(rows_v, buf_hbm.at[idx_v], sem).wait()

    buf = sc_scatter(scaled, sidx_flat)

    nblk = nrow // cap
    buf = pl.pallas_call(
        functools.partial(_ffn_body, nblk),
        grid=(e,),
        in_specs=[
            pl.BlockSpec((cap, d), lambda i: (i, 0)),
            pl.BlockSpec((1, d, d_ff), lambda i: (i, 0, 0)),
            pl.BlockSpec((1, d_ff, d), lambda i: (i, 0, 0)),
        ],
        out_specs=pl.BlockSpec((cap, d), lambda i: (e + i, 0)),
        out_shape=jax.ShapeDtypeStruct((nrow, d), jnp.float32),
        input_output_aliases={0: 0},
    )(buf, wi, wo)

    @functools.partial(
        pl.kernel,
        out_type=jax.ShapeDtypeStruct((n_tok, d), jnp.float32),
        mesh=mesh,
        scratch_types=[
            pltpu.VMEM((tpw, d), jnp.float32),
            pltpu.VMEM((tpw,), jnp.int32),
            pltpu.SemaphoreType.DMA,
        ],
    )
    def sc_gather(buf_hbm, gidx_hbm, out_hbm, rows_v, idx_v, sem):
        wid = lax.axis_index("s") * _NC + lax.axis_index("c")
        base = wid * tpw
        pltpu.sync_copy(gidx_hbm.at[pl.ds(base, tpw)], idx_v)
        pltpu.async_copy(buf_hbm.at[idx_v], rows_v, sem).wait()
        pltpu.sync_copy(rows_v, out_hbm.at[pl.ds(base, tpw)])

    out_flat = sc_gather(buf, gidx_flat)

    out = out_flat.reshape(b, s, d)
    router_logits = logits.reshape(b, s, e)
    final_expert_index = fei.reshape(b, s)
    return out, router_logits, final_expert_index
